# baseline (speedup 1.0000x reference)
import jax
import jax.numpy as jnp
from jax import lax
from jax.experimental import pallas as pl
from jax.experimental.pallas import tpu as pltpu
from jax.experimental.pallas import tpu_sc as plsc

NC = 2
NS = 16
CHUNK = 128
GK = 8


def _make_aggregate(n, e_pad, d, with_deg):
    nw = NC * NS
    n_chunks = e_pad // CHUNK
    cpw = n_chunks // nw
    ng = cpw // GK
    n_acc = n
    rpt = n // NS
    zpt = n_acc // NS
    assert n % NS == 0 and n_chunks % nw == 0 and cpw % GK == 0 and ng >= 2

    out_type = [jax.ShapeDtypeStruct((2 * n, d), jnp.float32)]
    scratch = [
        [pltpu.VMEM((GK, CHUNK), jnp.int32) for _ in range(2)],
        [pltpu.VMEM((GK, CHUNK), jnp.int32) for _ in range(2)],
        [pltpu.VMEM((CHUNK, d), jnp.float32) for _ in range(2)],
        pltpu.VMEM_SHARED((n_acc, d), jnp.float32),
        [pltpu.SemaphoreType.DMA for _ in range(2)],
        [pltpu.SemaphoreType.DMA for _ in range(2)],
        [pltpu.SemaphoreType.DMA for _ in range(2)],
    ]
    if with_deg:
        out_type.append(jax.ShapeDtypeStruct((2 * n, 16), jnp.float32))
        scratch += [
            pltpu.VMEM((CHUNK, 16), jnp.float32),
            pltpu.VMEM_SHARED((n_acc, 16), jnp.float32),
            [pltpu.SemaphoreType.DMA for _ in range(2)],
        ]

    mesh = plsc.VectorSubcoreMesh(core_axis_name="c", subcore_axis_name="s")

    def body(feat_hbm, src_hbm, dst_hbm, zacc_hbm, *rest):
        if with_deg:
            (zdeg_hbm, ones_hbm, out_hbm, deg_out,
             srcg, dstg, rows, acc_sh, sem_i, sem_g, sem_s,
             ones_v, deg_sh, sem_d) = rest
        else:
            (out_hbm, srcg, dstg, rows, acc_sh, sem_i, sem_g, sem_s) = rest
        c = lax.axis_index("c")
        s = lax.axis_index("s")
        w = s * NC + c

        pltpu.sync_copy(zacc_hbm, acc_sh.at[pl.ds(s * zpt, zpt)])
        if with_deg:
            pltpu.sync_copy(zdeg_hbm, deg_sh.at[pl.ds(s * zpt, zpt)])
            pltpu.sync_copy(ones_hbm, ones_v)
        plsc.subcore_barrier()

        def idx_fetch(g):
            p = g % 2
            base = (g * nw + w) * GK
            pltpu.async_copy(src_hbm.at[pl.ds(base, GK)], srcg[p], sem_i[p])
            pltpu.async_copy(dst_hbm.at[pl.ds(base, GK)], dstg[p], sem_i[p])

        def idx_wait(g):
            p = g % 2
            pltpu.make_async_copy(src_hbm.at[pl.ds(0, GK)], srcg[p],
                                  sem_i[p]).wait()
            pltpu.make_async_copy(dst_hbm.at[pl.ds(0, GK)], dstg[p],
                                  sem_i[p]).wait()

        def gather(i):
            g, k, b = i // GK, i % GK, i % 2
            pltpu.async_copy(feat_hbm.at[srcg[g % 2].at[k]], rows[b], sem_g[b])

        def gather_wait(i):
            b = i % 2
            pltpu.make_async_copy(feat_hbm.at[srcg[0].at[0]], rows[b],
                                  sem_g[b]).wait()

        def scatter(i):
            g, k, b = i // GK, i % GK, i % 2
            pltpu.async_copy(rows[b], acc_sh.at[dstg[g % 2].at[k]], sem_s[b],
                             add=True)
            if with_deg:
                pltpu.async_copy(ones_v, deg_sh.at[dstg[g % 2].at[k]],
                                 sem_d[b], add=True)

        def scatter_wait(i):
            b = i % 2
            pltpu.make_async_copy(rows[b], acc_sh.at[dstg[0].at[0]],
                                  sem_s[b]).wait()
            if with_deg:
                pltpu.make_async_copy(ones_v, deg_sh.at[dstg[0].at[0]],
                                      sem_d[b]).wait()

        idx_fetch(0)
        idx_fetch(1)
        idx_wait(0)
        gather(0)
        for i in range(cpw):
            gather_wait(i)
            scatter(i)
            nxt = i + 1
            if nxt < cpw:
                if nxt % GK == 0:
                    idx_wait(nxt // GK)
                if nxt >= 2:
                    scatter_wait(nxt - 2)
                if nxt % GK == 1 and 2 <= nxt // GK + 1 < ng:
                    idx_fetch(nxt // GK + 1)
                gather(nxt)
        scatter_wait(cpw - 2)
        scatter_wait(cpw - 1)
        plsc.subcore_barrier()

        pltpu.sync_copy(acc_sh.at[pl.ds(s * rpt, rpt)],
                        out_hbm.at[pl.ds(c * n + s * rpt, rpt)])
        if with_deg:
            pltpu.sync_copy(deg_sh.at[pl.ds(s * rpt, rpt)],
                            deg_out.at[pl.ds(c * n + s * rpt, rpt)])

    return pl.kernel(
        body, out_type=out_type, mesh=mesh, scratch_types=scratch,
        compiler_params=pltpu.CompilerParams(use_tc_tiling_on_sc=False))


def _make_dense(n, d, act, pad_cnt):
    blk = 1000
    assert n % blk == 0 and pad_cnt <= n
    grid = (n // blk,)

    def body(p0_r, p1_r, d0_r, d1_r, x_r, f0_r, wl_r, b_r, wr_r, o_r):
        base = pl.program_id(0) * blk
        rowid = base + lax.broadcasted_iota(jnp.int32, (blk, 1), 0)
        padm = (rowid < pad_cnt).astype(jnp.float32)
        deg = d0_r[:, :1] + d1_r[:, :1] - padm
        agg = (p0_r[...] + p1_r[...] - padm * f0_r[...]) / jnp.maximum(deg, 1.0)
        h = (jnp.dot(agg, wl_r[...], preferred_element_type=jnp.float32,
                     precision=lax.Precision.HIGHEST)
             + b_r[...]
             + jnp.dot(x_r[...], wr_r[...], preferred_element_type=jnp.float32,
                       precision=lax.Precision.HIGHEST))
        if act == "relu":
            o_r[...] = jnp.maximum(h, 0.0)
        else:
            m = jnp.max(h, axis=1, keepdims=True)
            ex = jnp.exp(h - m)
            o_r[...] = ex / jnp.sum(ex, axis=1, keepdims=True)

    row_spec = pl.BlockSpec((blk, d), lambda i: (i, 0))
    deg_spec = pl.BlockSpec((blk, 16), lambda i: (i, 0))
    full_spec = pl.BlockSpec((d, d), lambda i: (0, 0))
    bias_spec = pl.BlockSpec((1, d), lambda i: (0, 0))
    return pl.pallas_call(
        body,
        grid=grid,
        in_specs=[row_spec, row_spec, deg_spec, deg_spec, row_spec,
                  bias_spec, full_spec, bias_spec, full_spec],
        out_specs=row_spec,
        out_shape=jax.ShapeDtypeStruct((n, d), jnp.float32),
    )


def kernel(x, edge_index, W1_l, b1, W1_r, W2_l, b2, W2_r):
    n, d = x.shape
    e = edge_index.shape[1]
    nw = NC * NS
    quantum = CHUNK * nw * GK
    e_pad = ((e + quantum - 1) // quantum) * quantum
    src1d = jnp.concatenate(
        [edge_index[0], jnp.zeros((e_pad - e,), jnp.int32)])
    pad_cnt = e_pad - e
    assert pad_cnt <= n
    dst1d = jnp.concatenate(
        [edge_index[1], jnp.arange(pad_cnt, dtype=jnp.int32)])
    src2d = src1d.reshape(e_pad // CHUNK, CHUNK)
    dst2d = dst1d.reshape(e_pad // CHUNK, CHUNK)
    zacc = jnp.zeros((n // NS, d), jnp.float32)
    zdeg = jnp.zeros((n // NS, 16), jnp.float32)
    ones = jnp.ones((CHUNK, 16), jnp.float32)

    agg1, deg = _make_aggregate(n, e_pad, d, True)(
        x, src2d, dst2d, zacc, zdeg, ones)
    d0, d1 = deg[:n], deg[n:]
    h = _make_dense(n, d, "relu", pad_cnt)(
        agg1[:n], agg1[n:], d0, d1, x, x[0:1], W1_l.T, b1.reshape(1, -1),
        W1_r.T)
    agg2 = _make_aggregate(n, e_pad, d, False)(h, src2d, dst2d, zacc)
    out = _make_dense(n, d, "softmax", pad_cnt)(
        agg2[0][:n], agg2[0][n:], d0, d1, h, h[0:1], W2_l.T,
        b2.reshape(1, -1), W2_r.T)
    return out


# speedup vs baseline: 1.0025x; 1.0025x over previous
import jax
import jax.numpy as jnp
from jax import lax
from jax.experimental import pallas as pl
from jax.experimental.pallas import tpu as pltpu
from jax.experimental.pallas import tpu_sc as plsc

NC = 2
NS = 16
CHUNK = 128
GK = 8


def _make_aggregate(n, e_pad, d, with_deg):
    nw = NC * NS
    n_chunks = e_pad // CHUNK
    cpw = n_chunks // nw
    ng = cpw // GK
    n_acc = n
    rpt = n // NS
    zpt = n_acc // NS
    assert n % NS == 0 and n_chunks % nw == 0 and cpw % GK == 0 and ng >= 2

    out_type = [jax.ShapeDtypeStruct((2 * n, d), jnp.float32)]
    scratch = [
        [pltpu.VMEM((GK, CHUNK), jnp.int32) for _ in range(2)],
        [pltpu.VMEM((GK, CHUNK), jnp.int32) for _ in range(2)],
        [pltpu.VMEM((CHUNK, d), jnp.float32) for _ in range(2)],
        pltpu.VMEM_SHARED((n_acc, d), jnp.float32),
        [pltpu.SemaphoreType.DMA for _ in range(2)],
        [pltpu.SemaphoreType.DMA for _ in range(2)],
        [pltpu.SemaphoreType.DMA for _ in range(2)],
    ]
    if with_deg:
        out_type.append(jax.ShapeDtypeStruct((2 * n, 16), jnp.float32))
        scratch += [
            pltpu.VMEM((CHUNK, 16), jnp.float32),
            pltpu.VMEM_SHARED((n_acc, 16), jnp.float32),
            [pltpu.SemaphoreType.DMA for _ in range(2)],
        ]

    mesh = plsc.VectorSubcoreMesh(core_axis_name="c", subcore_axis_name="s")

    def body(feat_hbm, src_hbm, dst_hbm, zacc_hbm, *rest):
        if with_deg:
            (zdeg_hbm, ones_hbm, out_hbm, deg_out,
             srcg, dstg, rows, acc_sh, sem_i, sem_g, sem_s,
             ones_v, deg_sh, sem_d) = rest
        else:
            (out_hbm, srcg, dstg, rows, acc_sh, sem_i, sem_g, sem_s) = rest
        c = lax.axis_index("c")
        s = lax.axis_index("s")
        w = s * NC + c

        pltpu.sync_copy(zacc_hbm, acc_sh.at[pl.ds(s * zpt, zpt)])
        if with_deg:
            pltpu.sync_copy(zdeg_hbm, deg_sh.at[pl.ds(s * zpt, zpt)])
            pltpu.sync_copy(ones_hbm, ones_v)
        plsc.subcore_barrier()

        def idx_fetch(g, p):
            base = (g * nw + w) * GK
            pltpu.async_copy(src_hbm.at[pl.ds(base, GK)], srcg[p], sem_i[p])
            pltpu.async_copy(dst_hbm.at[pl.ds(base, GK)], dstg[p], sem_i[p])

        def idx_wait(p):
            pltpu.make_async_copy(src_hbm.at[pl.ds(0, GK)], srcg[p],
                                  sem_i[p]).wait()
            pltpu.make_async_copy(dst_hbm.at[pl.ds(0, GK)], dstg[p],
                                  sem_i[p]).wait()

        def gather(kr, pg, b):
            pltpu.async_copy(feat_hbm.at[srcg[pg].at[kr]], rows[b], sem_g[b])

        def gather_wait(b):
            pltpu.make_async_copy(feat_hbm.at[srcg[0].at[0]], rows[b],
                                  sem_g[b]).wait()

        def scatter(kr, pg, b):
            pltpu.async_copy(rows[b], acc_sh.at[dstg[pg].at[kr]], sem_s[b],
                             add=True)
            if with_deg:
                pltpu.async_copy(ones_v, deg_sh.at[dstg[pg].at[kr]],
                                 sem_d[b], add=True)

        def scatter_wait(b):
            pltpu.make_async_copy(rows[b], acc_sh.at[dstg[0].at[0]],
                                  sem_s[b]).wait()
            if with_deg:
                pltpu.make_async_copy(ones_v, deg_sh.at[dstg[0].at[0]],
                                      sem_d[b]).wait()

        nj = ng // 2
        assert cpw == 16 * nj
        idx_fetch(0, 0)
        idx_wait(0)
        gather(0, 0, 0)

        def pair_body(j, carry):
            g0 = 2 * j
            for k in range(16):
                b = k % 2
                pg = k // 8
                kr = k % 8
                gather_wait(b)
                scatter(kr, pg, b)
                if k == 15:
                    scatter_wait(0)

                    @pl.when(j < nj - 1)
                    def _():
                        idx_wait(0)
                        gather(0, 0, 0)

                    @pl.when(j == nj - 1)
                    def _():
                        scatter_wait(1)
                else:
                    if k == 7:
                        idx_wait(1)
                    if k == 0:
                        @pl.when(j > 0)
                        def _():
                            scatter_wait(1)
                    else:
                        scatter_wait((k - 1) % 2)
                    if k == 1:
                        idx_fetch(g0 + 1, 1)
                    if k == 9:
                        @pl.when(j < nj - 1)
                        def _():
                            idx_fetch(g0 + 2, 0)
                    gather((k + 1) % 8, (k + 1) // 8 % 2, (k + 1) % 2)
            return carry

        lax.fori_loop(0, nj, pair_body, 0)
        plsc.subcore_barrier()

        pltpu.sync_copy(acc_sh.at[pl.ds(s * rpt, rpt)],
                        out_hbm.at[pl.ds(c * n + s * rpt, rpt)])
        if with_deg:
            pltpu.sync_copy(deg_sh.at[pl.ds(s * rpt, rpt)],
                            deg_out.at[pl.ds(c * n + s * rpt, rpt)])

    return pl.kernel(
        body, out_type=out_type, mesh=mesh, scratch_types=scratch,
        compiler_params=pltpu.CompilerParams(use_tc_tiling_on_sc=False))


def _make_dense(n, d, act, pad_cnt):
    blk = 1000
    assert n % blk == 0 and pad_cnt <= n
    grid = (n // blk,)

    def body(p0_r, p1_r, d0_r, d1_r, x_r, f0_r, wl_r, b_r, wr_r, o_r):
        base = pl.program_id(0) * blk
        rowid = base + lax.broadcasted_iota(jnp.int32, (blk, 1), 0)
        padm = (rowid < pad_cnt).astype(jnp.float32)
        deg = d0_r[:, :1] + d1_r[:, :1] - padm
        agg = (p0_r[...] + p1_r[...] - padm * f0_r[...]) / jnp.maximum(deg, 1.0)
        h = (jnp.dot(agg, wl_r[...], preferred_element_type=jnp.float32,
                     precision=lax.Precision.HIGHEST)
             + b_r[...]
             + jnp.dot(x_r[...], wr_r[...], preferred_element_type=jnp.float32,
                       precision=lax.Precision.HIGHEST))
        if act == "relu":
            o_r[...] = jnp.maximum(h, 0.0)
        else:
            m = jnp.max(h, axis=1, keepdims=True)
            ex = jnp.exp(h - m)
            o_r[...] = ex / jnp.sum(ex, axis=1, keepdims=True)

    row_spec = pl.BlockSpec((blk, d), lambda i: (i, 0))
    deg_spec = pl.BlockSpec((blk, 16), lambda i: (i, 0))
    full_spec = pl.BlockSpec((d, d), lambda i: (0, 0))
    bias_spec = pl.BlockSpec((1, d), lambda i: (0, 0))
    return pl.pallas_call(
        body,
        grid=grid,
        in_specs=[row_spec, row_spec, deg_spec, deg_spec, row_spec,
                  bias_spec, full_spec, bias_spec, full_spec],
        out_specs=row_spec,
        out_shape=jax.ShapeDtypeStruct((n, d), jnp.float32),
    )


def kernel(x, edge_index, W1_l, b1, W1_r, W2_l, b2, W2_r):
    n, d = x.shape
    e = edge_index.shape[1]
    nw = NC * NS
    quantum = CHUNK * nw * GK
    e_pad = ((e + quantum - 1) // quantum) * quantum
    src1d = jnp.concatenate(
        [edge_index[0], jnp.zeros((e_pad - e,), jnp.int32)])
    pad_cnt = e_pad - e
    assert pad_cnt <= n
    dst1d = jnp.concatenate(
        [edge_index[1], jnp.arange(pad_cnt, dtype=jnp.int32)])
    src2d = src1d.reshape(e_pad // CHUNK, CHUNK)
    dst2d = dst1d.reshape(e_pad // CHUNK, CHUNK)
    zacc = jnp.zeros((n // NS, d), jnp.float32)
    zdeg = jnp.zeros((n // NS, 16), jnp.float32)
    ones = jnp.ones((CHUNK, 16), jnp.float32)

    agg1, deg = _make_aggregate(n, e_pad, d, True)(
        x, src2d, dst2d, zacc, zdeg, ones)
    d0, d1 = deg[:n], deg[n:]
    h = _make_dense(n, d, "relu", pad_cnt)(
        agg1[:n], agg1[n:], d0, d1, x, x[0:1], W1_l.T, b1.reshape(1, -1),
        W1_r.T)
    agg2 = _make_aggregate(n, e_pad, d, False)(h, src2d, dst2d, zacc)
    out = _make_dense(n, d, "softmax", pad_cnt)(
        agg2[0][:n], agg2[0][n:], d0, d1, h, h[0:1], W2_l.T,
        b2.reshape(1, -1), W2_r.T)
    return out


# speedup vs baseline: 2.5307x; 2.5243x over previous
import jax
import jax.numpy as jnp
from jax import lax
from jax.experimental import pallas as pl
from jax.experimental.pallas import tpu as pltpu
from jax.experimental.pallas import tpu_sc as plsc

NC = 2
NS = 16
CHUNK = 128
GK = 8


def _make_aggregate(n, e_pad, d, with_deg):
    nw = NC * NS
    n_chunks = e_pad // CHUNK
    cpw = n_chunks // nw
    ng = cpw // GK
    n_acc = n
    rpt = n // NS
    zpt = n_acc // NS
    assert n % NS == 0 and n_chunks % nw == 0 and cpw % GK == 0 and ng >= 2

    out_type = [jax.ShapeDtypeStruct((2 * n, d), jnp.float32)]
    scratch = [
        [pltpu.VMEM((GK, CHUNK), jnp.int32) for _ in range(2)],
        [pltpu.VMEM((GK, CHUNK), jnp.int32) for _ in range(2)],
        [pltpu.VMEM((CHUNK, d), jnp.float32) for _ in range(2)],
        pltpu.VMEM_SHARED((n_acc, d), jnp.float32),
        [pltpu.SemaphoreType.DMA for _ in range(2)],
        [pltpu.SemaphoreType.DMA for _ in range(2)],
        [pltpu.SemaphoreType.DMA for _ in range(2)],
    ]
    if with_deg:
        out_type.append(jax.ShapeDtypeStruct((2 * n, 16), jnp.float32))
        scratch += [
            pltpu.VMEM((CHUNK, 16), jnp.float32),
            pltpu.VMEM_SHARED((n_acc, 16), jnp.float32),
            [pltpu.SemaphoreType.DMA for _ in range(2)],
        ]

    mesh = plsc.VectorSubcoreMesh(core_axis_name="c", subcore_axis_name="s")

    def body(feat_hbm, src_hbm, dst_hbm, zacc_hbm, *rest):
        if with_deg:
            (zdeg_hbm, ones_hbm, out_hbm, deg_out,
             srcg, dstg, rows, acc_sh, sem_i, sem_g, sem_s,
             ones_v, deg_sh, sem_d) = rest
        else:
            (out_hbm, srcg, dstg, rows, acc_sh, sem_i, sem_g, sem_s) = rest
        c = lax.axis_index("c")
        s = lax.axis_index("s")
        w = s * NC + c

        pltpu.sync_copy(zacc_hbm, acc_sh.at[pl.ds(s * zpt, zpt)])
        if with_deg:
            pltpu.sync_copy(zdeg_hbm, deg_sh.at[pl.ds(s * zpt, zpt)])
            pltpu.sync_copy(ones_hbm, ones_v)
        plsc.subcore_barrier()

        def idx_fetch(g, p):
            base = (g * nw + w) * GK
            pltpu.async_copy(src_hbm.at[pl.ds(base, GK)], srcg[p], sem_i[p])
            pltpu.async_copy(dst_hbm.at[pl.ds(base, GK)], dstg[p], sem_i[p])

        def idx_wait(p):
            pltpu.make_async_copy(src_hbm.at[pl.ds(0, GK)], srcg[p],
                                  sem_i[p]).wait()
            pltpu.make_async_copy(dst_hbm.at[pl.ds(0, GK)], dstg[p],
                                  sem_i[p]).wait()

        def gather(kr, pg, b):
            pltpu.async_copy(feat_hbm.at[srcg[pg].at[kr]], rows[b], sem_g[b])

        def gather_wait(b):
            pltpu.make_async_copy(feat_hbm.at[srcg[0].at[0]], rows[b],
                                  sem_g[b]).wait()

        def scatter(kr, pg, b):
            pltpu.async_copy(rows[b], acc_sh.at[dstg[pg].at[kr]], sem_s[b],
                             add=True)
            if with_deg:
                pltpu.async_copy(ones_v, deg_sh.at[dstg[pg].at[kr]],
                                 sem_d[b], add=True)

        def scatter_wait(b):
            pltpu.make_async_copy(rows[b], acc_sh.at[dstg[0].at[0]],
                                  sem_s[b]).wait()
            if with_deg:
                pltpu.make_async_copy(ones_v, deg_sh.at[dstg[0].at[0]],
                                      sem_d[b]).wait()

        nj = ng // 2
        assert cpw == 16 * nj
        idx_fetch(0, 0)
        idx_wait(0)
        gather(0, 0, 0)

        def pair_body(j, carry):
            g0 = 2 * j
            for k in range(16):
                b = k % 2
                pg = k // 8
                kr = k % 8
                gather_wait(b)
                scatter(kr, pg, b)
                if k == 15:
                    scatter_wait(0)

                    @pl.when(j < nj - 1)
                    def _():
                        idx_wait(0)
                        gather(0, 0, 0)

                    @pl.when(j == nj - 1)
                    def _():
                        scatter_wait(1)
                else:
                    if k == 7:
                        idx_wait(1)
                    if k == 0:
                        @pl.when(j > 0)
                        def _():
                            scatter_wait(1)
                    else:
                        scatter_wait((k - 1) % 2)
                    if k == 1:
                        idx_fetch(g0 + 1, 1)
                    if k == 9:
                        @pl.when(j < nj - 1)
                        def _():
                            idx_fetch(g0 + 2, 0)
                    gather((k + 1) % 8, (k + 1) // 8 % 2, (k + 1) % 2)
            return carry

        lax.fori_loop(0, nj, pair_body, 0)
        plsc.subcore_barrier()

        pltpu.sync_copy(acc_sh.at[pl.ds(s * rpt, rpt)],
                        out_hbm.at[pl.ds(c * n + s * rpt, rpt)])
        if with_deg:
            pltpu.sync_copy(deg_sh.at[pl.ds(s * rpt, rpt)],
                            deg_out.at[pl.ds(c * n + s * rpt, rpt)])

    return pl.kernel(
        body, out_type=out_type, mesh=mesh, scratch_types=scratch,
        compiler_params=pltpu.CompilerParams(use_tc_tiling_on_sc=False))


def _make_dense(n, d, act, pad_cnt):
    blk = 1000
    assert n % blk == 0 and pad_cnt <= n
    grid = (n // blk,)

    def body(p0_r, p1_r, d0_r, d1_r, x_r, wl_r, b_r, wr_r, o_r):
        base = pl.program_id(0) * blk
        rowid = base + lax.broadcasted_iota(jnp.int32, (blk, 1), 0)
        padm = (rowid < pad_cnt).astype(jnp.float32)
        deg = d0_r[:, :1] + d1_r[:, :1] - padm
        agg = (p0_r[...] + p1_r[...] - padm * x_r[...]) / jnp.maximum(deg, 1.0)
        h = (jnp.dot(agg, wl_r[...], preferred_element_type=jnp.float32,
                     precision=lax.Precision.HIGHEST)
             + b_r[...]
             + jnp.dot(x_r[...], wr_r[...], preferred_element_type=jnp.float32,
                       precision=lax.Precision.HIGHEST))
        if act == "relu":
            o_r[...] = jnp.maximum(h, 0.0)
        else:
            m = jnp.max(h, axis=1, keepdims=True)
            ex = jnp.exp(h - m)
            o_r[...] = ex / jnp.sum(ex, axis=1, keepdims=True)

    row_spec = pl.BlockSpec((blk, d), lambda i: (i, 0))
    deg_spec = pl.BlockSpec((blk, 16), lambda i: (i, 0))
    full_spec = pl.BlockSpec((d, d), lambda i: (0, 0))
    bias_spec = pl.BlockSpec((1, d), lambda i: (0, 0))
    return pl.pallas_call(
        body,
        grid=grid,
        in_specs=[row_spec, row_spec, deg_spec, deg_spec, row_spec,
                  full_spec, bias_spec, full_spec],
        out_specs=row_spec,
        out_shape=jax.ShapeDtypeStruct((n, d), jnp.float32),
    )


def kernel(x, edge_index, W1_l, b1, W1_r, W2_l, b2, W2_r):
    n, d = x.shape
    e = edge_index.shape[1]
    nw = NC * NS
    quantum = CHUNK * nw * GK
    e_pad = ((e + quantum - 1) // quantum) * quantum
    src1d = jnp.concatenate(
        [edge_index[0], jnp.arange(e_pad - e, dtype=jnp.int32)])
    pad_cnt = e_pad - e
    assert pad_cnt <= n
    dst1d = jnp.concatenate(
        [edge_index[1], jnp.arange(pad_cnt, dtype=jnp.int32)])
    src2d = src1d.reshape(e_pad // CHUNK, CHUNK)
    dst2d = dst1d.reshape(e_pad // CHUNK, CHUNK)
    zacc = jnp.zeros((n // NS, d), jnp.float32)
    zdeg = jnp.zeros((n // NS, 16), jnp.float32)
    ones = jnp.ones((CHUNK, 16), jnp.float32)

    agg1, deg = _make_aggregate(n, e_pad, d, True)(
        x, src2d, dst2d, zacc, zdeg, ones)
    d0, d1 = deg[:n], deg[n:]
    h = _make_dense(n, d, "relu", pad_cnt)(
        agg1[:n], agg1[n:], d0, d1, x, W1_l.T, b1.reshape(1, -1), W1_r.T)
    agg2 = _make_aggregate(n, e_pad, d, False)(h, src2d, dst2d, zacc)
    out = _make_dense(n, d, "softmax", pad_cnt)(
        agg2[0][:n], agg2[0][n:], d0, d1, h, W2_l.T, b2.reshape(1, -1),
        W2_r.T)
    return out


# speedup vs baseline: 2.5630x; 1.0128x over previous
import jax
import jax.numpy as jnp
from jax import lax
from jax.experimental import pallas as pl
from jax.experimental.pallas import tpu as pltpu
from jax.experimental.pallas import tpu_sc as plsc

NC = 2
NS = 16
CHUNK = 128
GK = 8


def _make_aggregate(n, e_pad, d, with_deg):
    nw = NC * NS
    n_chunks = e_pad // CHUNK
    cpw = n_chunks // nw
    ng = cpw // GK
    n_acc = n
    rpt = n // NS
    zpt = n_acc // NS
    assert n % NS == 0 and n_chunks % nw == 0 and cpw % GK == 0 and ng >= 2

    out_type = [jax.ShapeDtypeStruct((2 * n, d), jnp.float32)]
    scratch = [
        [pltpu.VMEM((GK, CHUNK), jnp.int32) for _ in range(2)],
        [pltpu.VMEM((GK, CHUNK), jnp.int32) for _ in range(2)],
        [pltpu.VMEM((CHUNK, d), jnp.float32) for _ in range(2)],
        pltpu.VMEM_SHARED((n_acc, d), jnp.float32),
        [pltpu.SemaphoreType.DMA for _ in range(2)],
        [pltpu.SemaphoreType.DMA for _ in range(2)],
        [pltpu.SemaphoreType.DMA for _ in range(2)],
    ]
    if with_deg:
        out_type.append(jax.ShapeDtypeStruct((2 * n, 16), jnp.float32))
        scratch += [
            pltpu.VMEM((CHUNK, 16), jnp.float32),
            pltpu.VMEM_SHARED((n_acc, 16), jnp.float32),
            [pltpu.SemaphoreType.DMA for _ in range(2)],
        ]

    mesh = plsc.VectorSubcoreMesh(core_axis_name="c", subcore_axis_name="s")

    def body(feat_hbm, src_hbm, dst_hbm, zacc_hbm, *rest):
        if with_deg:
            (zdeg_hbm, ones_hbm, out_hbm, deg_out,
             srcg, dstg, rows, acc_sh, sem_i, sem_g, sem_s,
             ones_v, deg_sh, sem_d) = rest
        else:
            (out_hbm, srcg, dstg, rows, acc_sh, sem_i, sem_g, sem_s) = rest
        c = lax.axis_index("c")
        s = lax.axis_index("s")
        w = s * NC + c

        pltpu.sync_copy(zacc_hbm, acc_sh.at[pl.ds(s * zpt, zpt)])
        if with_deg:
            pltpu.sync_copy(zdeg_hbm, deg_sh.at[pl.ds(s * zpt, zpt)])
            pltpu.sync_copy(ones_hbm, ones_v)
        plsc.subcore_barrier()

        def idx_fetch(g, p):
            base = (g * nw + w) * GK
            pltpu.async_copy(src_hbm.at[pl.ds(base, GK)], srcg[p], sem_i[p])
            pltpu.async_copy(dst_hbm.at[pl.ds(base, GK)], dstg[p], sem_i[p])

        def idx_wait(p):
            pltpu.make_async_copy(src_hbm.at[pl.ds(0, GK)], srcg[p],
                                  sem_i[p]).wait()
            pltpu.make_async_copy(dst_hbm.at[pl.ds(0, GK)], dstg[p],
                                  sem_i[p]).wait()

        def gather(kr, pg, b):
            pltpu.async_copy(feat_hbm.at[srcg[pg].at[kr]], rows[b], sem_g[b])

        def gather_wait(b):
            pltpu.make_async_copy(feat_hbm.at[srcg[0].at[0]], rows[b],
                                  sem_g[b]).wait()

        def scatter(kr, pg, b):
            pltpu.async_copy(rows[b], acc_sh.at[dstg[pg].at[kr]], sem_s[b],
                             add=True)
            if with_deg:
                pltpu.async_copy(ones_v, deg_sh.at[dstg[pg].at[kr]],
                                 sem_d[b], add=True)

        def scatter_wait(b):
            pltpu.make_async_copy(rows[b], acc_sh.at[dstg[0].at[0]],
                                  sem_s[b]).wait()
            if with_deg:
                pltpu.make_async_copy(ones_v, deg_sh.at[dstg[0].at[0]],
                                      sem_d[b]).wait()

        nj = ng // 2
        assert cpw == 16 * nj
        idx_fetch(0, 0)
        idx_wait(0)
        gather(0, 0, 0)

        def pair_body(j, carry):
            g0 = 2 * j
            for k in range(16):
                b = k % 2
                pg = k // 8
                kr = k % 8
                gather_wait(b)
                scatter(kr, pg, b)
                if k == 15:
                    scatter_wait(0)

                    @pl.when(j < nj - 1)
                    def _():
                        idx_wait(0)
                        gather(0, 0, 0)

                    @pl.when(j == nj - 1)
                    def _():
                        scatter_wait(1)
                else:
                    if k == 7:
                        idx_wait(1)
                    if k == 0:
                        @pl.when(j > 0)
                        def _():
                            scatter_wait(1)
                    else:
                        scatter_wait((k - 1) % 2)
                    if k == 1:
                        idx_fetch(g0 + 1, 1)
                    if k == 9:
                        @pl.when(j < nj - 1)
                        def _():
                            idx_fetch(g0 + 2, 0)
                    gather((k + 1) % 8, (k + 1) // 8 % 2, (k + 1) % 2)
            return carry

        lax.fori_loop(0, nj, pair_body, 0)
        plsc.subcore_barrier()

        pltpu.sync_copy(acc_sh.at[pl.ds(s * rpt, rpt)],
                        out_hbm.at[pl.ds(c * n + s * rpt, rpt)])
        if with_deg:
            pltpu.sync_copy(deg_sh.at[pl.ds(s * rpt, rpt)],
                            deg_out.at[pl.ds(c * n + s * rpt, rpt)])

    return pl.kernel(
        body, out_type=out_type, mesh=mesh, scratch_types=scratch,
        compiler_params=pltpu.CompilerParams(use_tc_tiling_on_sc=False))


def _make_xr(n, d):
    blk = 1000
    grid = (n // blk,)

    def body(x_r, wr_r, b_r, o_r):
        o_r[...] = (jnp.dot(x_r[...], wr_r[...],
                            preferred_element_type=jnp.float32,
                            precision=lax.Precision.HIGHEST) + b_r[...])

    row_spec = pl.BlockSpec((blk, d), lambda i: (i, 0))
    full_spec = pl.BlockSpec((d, d), lambda i: (0, 0))
    bias_spec = pl.BlockSpec((1, d), lambda i: (0, 0))
    return pl.pallas_call(
        body, grid=grid,
        in_specs=[row_spec, full_spec, bias_spec],
        out_specs=row_spec,
        out_shape=jax.ShapeDtypeStruct((n, d), jnp.float32),
    )


def _make_dense(n, d, act, pad_cnt):
    blk = 1000
    assert n % blk == 0 and pad_cnt <= n
    grid = (n // blk,)

    def body(p0_r, p1_r, d0_r, d1_r, x_r, wl_r, xr_r, o_r):
        base = pl.program_id(0) * blk
        rowid = base + lax.broadcasted_iota(jnp.int32, (blk, 1), 0)
        padm = (rowid < pad_cnt).astype(jnp.float32)
        deg = d0_r[:, :1] + d1_r[:, :1] - padm
        agg = (p0_r[...] + p1_r[...] - padm * x_r[...]) / jnp.maximum(deg, 1.0)
        h = (jnp.dot(agg, wl_r[...], preferred_element_type=jnp.float32,
                     precision=lax.Precision.HIGHEST)
             + xr_r[...])
        if act == "relu":
            o_r[...] = jnp.maximum(h, 0.0)
        else:
            m = jnp.max(h, axis=1, keepdims=True)
            ex = jnp.exp(h - m)
            o_r[...] = ex / jnp.sum(ex, axis=1, keepdims=True)

    row_spec = pl.BlockSpec((blk, d), lambda i: (i, 0))
    deg_spec = pl.BlockSpec((blk, 16), lambda i: (i, 0))
    full_spec = pl.BlockSpec((d, d), lambda i: (0, 0))
    bias_spec = pl.BlockSpec((1, d), lambda i: (0, 0))
    return pl.pallas_call(
        body,
        grid=grid,
        in_specs=[row_spec, row_spec, deg_spec, deg_spec, row_spec,
                  full_spec, row_spec],
        out_specs=row_spec,
        out_shape=jax.ShapeDtypeStruct((n, d), jnp.float32),
    )


def kernel(x, edge_index, W1_l, b1, W1_r, W2_l, b2, W2_r):
    n, d = x.shape
    e = edge_index.shape[1]
    nw = NC * NS
    quantum = CHUNK * nw * GK
    e_pad = ((e + quantum - 1) // quantum) * quantum
    src1d = jnp.concatenate(
        [edge_index[0], jnp.arange(e_pad - e, dtype=jnp.int32)])
    pad_cnt = e_pad - e
    assert pad_cnt <= n
    dst1d = jnp.concatenate(
        [edge_index[1], jnp.arange(pad_cnt, dtype=jnp.int32)])
    src2d = src1d.reshape(e_pad // CHUNK, CHUNK)
    dst2d = dst1d.reshape(e_pad // CHUNK, CHUNK)
    zacc = jnp.zeros((n // NS, d), jnp.float32)
    zdeg = jnp.zeros((n // NS, 16), jnp.float32)
    ones = jnp.ones((CHUNK, 16), jnp.float32)

    xr_fn = _make_xr(n, d)
    xr1 = xr_fn(x, W1_r.T, b1.reshape(1, -1))
    agg1, deg = _make_aggregate(n, e_pad, d, True)(
        x, src2d, dst2d, zacc, zdeg, ones)
    d0, d1 = deg[:n], deg[n:]
    h = _make_dense(n, d, "relu", pad_cnt)(
        agg1[:n], agg1[n:], d0, d1, x, W1_l.T, xr1)
    xr2 = xr_fn(h, W2_r.T, b2.reshape(1, -1))
    agg2 = _make_aggregate(n, e_pad, d, False)(h, src2d, dst2d, zacc)
    out = _make_dense(n, d, "softmax", pad_cnt)(
        agg2[0][:n], agg2[0][n:], d0, d1, h, W2_l.T, xr2)
    return out


# speedup vs baseline: 2.5694x; 1.0025x over previous
import jax
import jax.numpy as jnp
from jax import lax
from jax.experimental import pallas as pl
from jax.experimental.pallas import tpu as pltpu
from jax.experimental.pallas import tpu_sc as plsc

NC = 2
NS = 16
CHUNK = 128
GK = 8


def _make_aggregate(n, e_pad, d, with_deg):
    nw = NC * NS
    n_chunks = e_pad // CHUNK
    cpw = n_chunks // nw
    ng = cpw // GK
    n_acc = n
    rpt = n // NS
    zpt = n_acc // NS
    assert n % NS == 0 and n_chunks % nw == 0 and cpw % GK == 0 and ng >= 2

    out_type = [jax.ShapeDtypeStruct((2 * n, d), jnp.float32)]
    scratch = [
        [pltpu.VMEM((GK, CHUNK), jnp.int32) for _ in range(2)],
        [pltpu.VMEM((GK, CHUNK), jnp.int32) for _ in range(2)],
        [pltpu.VMEM((CHUNK, d), jnp.float32) for _ in range(2)],
        pltpu.VMEM_SHARED((n_acc, d), jnp.float32),
        [pltpu.SemaphoreType.DMA for _ in range(2)],
        [pltpu.SemaphoreType.DMA for _ in range(2)],
        [pltpu.SemaphoreType.DMA for _ in range(2)],
    ]
    if with_deg:
        out_type.append(jax.ShapeDtypeStruct((2 * n, 16), jnp.float32))
        scratch += [
            pltpu.VMEM((CHUNK, 16), jnp.float32),
            pltpu.VMEM_SHARED((n_acc, 16), jnp.float32),
            [pltpu.SemaphoreType.DMA for _ in range(2)],
        ]

    mesh = plsc.VectorSubcoreMesh(core_axis_name="c", subcore_axis_name="s")

    def body(feat_hbm, src_hbm, dst_hbm, zacc_hbm, *rest):
        if with_deg:
            (zdeg_hbm, ones_hbm, out_hbm, deg_out,
             srcg, dstg, rows, acc_sh, sem_i, sem_g, sem_s,
             ones_v, deg_sh, sem_d) = rest
        else:
            (out_hbm, srcg, dstg, rows, acc_sh, sem_i, sem_g, sem_s) = rest
        c = lax.axis_index("c")
        s = lax.axis_index("s")
        w = s * NC + c

        pltpu.sync_copy(zacc_hbm.at[pl.ds(s * zpt, zpt)],
                        acc_sh.at[pl.ds(s * zpt, zpt)])
        if with_deg:
            pltpu.sync_copy(zdeg_hbm.at[pl.ds(s * zpt, zpt)],
                            deg_sh.at[pl.ds(s * zpt, zpt)])
            pltpu.sync_copy(ones_hbm, ones_v)
        plsc.subcore_barrier()

        def idx_fetch(g, p):
            base = (g * nw + w) * GK
            pltpu.async_copy(src_hbm.at[pl.ds(base, GK)], srcg[p], sem_i[p])
            pltpu.async_copy(dst_hbm.at[pl.ds(base, GK)], dstg[p], sem_i[p])

        def idx_wait(p):
            pltpu.make_async_copy(src_hbm.at[pl.ds(0, GK)], srcg[p],
                                  sem_i[p]).wait()
            pltpu.make_async_copy(dst_hbm.at[pl.ds(0, GK)], dstg[p],
                                  sem_i[p]).wait()

        def gather(kr, pg, b):
            pltpu.async_copy(feat_hbm.at[srcg[pg].at[kr]], rows[b], sem_g[b])

        def gather_wait(b):
            pltpu.make_async_copy(feat_hbm.at[srcg[0].at[0]], rows[b],
                                  sem_g[b]).wait()

        def scatter(kr, pg, b):
            pltpu.async_copy(rows[b], acc_sh.at[dstg[pg].at[kr]], sem_s[b],
                             add=True)
            if with_deg:
                pltpu.async_copy(ones_v, deg_sh.at[dstg[pg].at[kr]],
                                 sem_d[b], add=True)

        def scatter_wait(b):
            pltpu.make_async_copy(rows[b], acc_sh.at[dstg[0].at[0]],
                                  sem_s[b]).wait()
            if with_deg:
                pltpu.make_async_copy(ones_v, deg_sh.at[dstg[0].at[0]],
                                      sem_d[b]).wait()

        nj = ng // 2
        assert cpw == 16 * nj
        idx_fetch(0, 0)
        idx_wait(0)
        gather(0, 0, 0)

        def pair_body(j, carry):
            g0 = 2 * j
            for k in range(16):
                b = k % 2
                pg = k // 8
                kr = k % 8
                gather_wait(b)
                scatter(kr, pg, b)
                if k == 15:
                    scatter_wait(0)

                    @pl.when(j < nj - 1)
                    def _():
                        idx_wait(0)
                        gather(0, 0, 0)

                    @pl.when(j == nj - 1)
                    def _():
                        scatter_wait(1)
                else:
                    if k == 7:
                        idx_wait(1)
                    if k == 0:
                        @pl.when(j > 0)
                        def _():
                            scatter_wait(1)
                    else:
                        scatter_wait((k - 1) % 2)
                    if k == 1:
                        idx_fetch(g0 + 1, 1)
                    if k == 9:
                        @pl.when(j < nj - 1)
                        def _():
                            idx_fetch(g0 + 2, 0)
                    gather((k + 1) % 8, (k + 1) // 8 % 2, (k + 1) % 2)
            return carry

        lax.fori_loop(0, nj, pair_body, 0)
        plsc.subcore_barrier()

        pltpu.sync_copy(acc_sh.at[pl.ds(s * rpt, rpt)],
                        out_hbm.at[pl.ds(c * n + s * rpt, rpt)])
        if with_deg:
            pltpu.sync_copy(deg_sh.at[pl.ds(s * rpt, rpt)],
                            deg_out.at[pl.ds(c * n + s * rpt, rpt)])

    return pl.kernel(
        body, out_type=out_type, mesh=mesh, scratch_types=scratch,
        compiler_params=pltpu.CompilerParams(use_tc_tiling_on_sc=False))


def _make_xr(n, d):
    blk = 1000
    grid = (n // blk,)

    def body(x_r, wr_r, b_r, o_r):
        o_r[...] = (jnp.dot(x_r[...], wr_r[...],
                            preferred_element_type=jnp.float32,
                            precision=lax.Precision.HIGHEST) + b_r[...])

    row_spec = pl.BlockSpec((blk, d), lambda i: (i, 0))
    full_spec = pl.BlockSpec((d, d), lambda i: (0, 0))
    bias_spec = pl.BlockSpec((1, d), lambda i: (0, 0))
    return pl.pallas_call(
        body, grid=grid,
        in_specs=[row_spec, full_spec, bias_spec],
        out_specs=row_spec,
        out_shape=jax.ShapeDtypeStruct((n, d), jnp.float32),
    )


def _make_dense(n, d, act, pad_cnt):
    blk = 1000
    assert n % blk == 0 and pad_cnt <= n
    grid = (n // blk,)

    def body(p0_r, p1_r, d0_r, d1_r, x_r, wl_r, xr_r, o_r):
        base = pl.program_id(0) * blk
        rowid = base + lax.broadcasted_iota(jnp.int32, (blk, 1), 0)
        padm = (rowid < pad_cnt).astype(jnp.float32)
        deg = d0_r[:, :1] + d1_r[:, :1] - padm
        agg = (p0_r[...] + p1_r[...] - padm * x_r[...]) / jnp.maximum(deg, 1.0)
        h = (jnp.dot(agg, wl_r[...], preferred_element_type=jnp.float32,
                     precision=lax.Precision.HIGHEST)
             + xr_r[...])
        if act == "relu":
            o_r[...] = jnp.maximum(h, 0.0)
        else:
            m = jnp.max(h, axis=1, keepdims=True)
            ex = jnp.exp(h - m)
            o_r[...] = ex / jnp.sum(ex, axis=1, keepdims=True)

    row_spec = pl.BlockSpec((blk, d), lambda i: (i, 0))
    deg_spec = pl.BlockSpec((blk, 16), lambda i: (i, 0))
    full_spec = pl.BlockSpec((d, d), lambda i: (0, 0))
    bias_spec = pl.BlockSpec((1, d), lambda i: (0, 0))
    return pl.pallas_call(
        body,
        grid=grid,
        in_specs=[row_spec, row_spec, deg_spec, deg_spec, row_spec,
                  full_spec, row_spec],
        out_specs=row_spec,
        out_shape=jax.ShapeDtypeStruct((n, d), jnp.float32),
    )


def kernel(x, edge_index, W1_l, b1, W1_r, W2_l, b2, W2_r):
    n, d = x.shape
    e = edge_index.shape[1]
    nw = NC * NS
    quantum = CHUNK * nw * GK
    e_pad = ((e + quantum - 1) // quantum) * quantum
    src1d = jnp.concatenate(
        [edge_index[0], jnp.arange(e_pad - e, dtype=jnp.int32)])
    pad_cnt = e_pad - e
    assert pad_cnt <= n
    dst1d = jnp.concatenate(
        [edge_index[1], jnp.arange(pad_cnt, dtype=jnp.int32)])
    src2d = src1d.reshape(e_pad // CHUNK, CHUNK)
    dst2d = dst1d.reshape(e_pad // CHUNK, CHUNK)
    zacc = jnp.zeros((n, d), jnp.float32)
    zdeg = jnp.zeros((n, 16), jnp.float32)
    ones = jnp.ones((CHUNK, 16), jnp.float32)

    xr_fn = _make_xr(n, d)
    xr1 = xr_fn(x, W1_r.T, b1.reshape(1, -1))
    agg1, deg = _make_aggregate(n, e_pad, d, True)(
        x, src2d, dst2d, zacc, zdeg, ones)
    d0, d1 = deg[:n], deg[n:]
    h = _make_dense(n, d, "relu", pad_cnt)(
        agg1[:n], agg1[n:], d0, d1, x, W1_l.T, xr1)
    xr2 = xr_fn(h, W2_r.T, b2.reshape(1, -1))
    agg2 = _make_aggregate(n, e_pad, d, False)(h, src2d, dst2d, zacc)
    out = _make_dense(n, d, "softmax", pad_cnt)(
        agg2[0][:n], agg2[0][n:], d0, d1, h, W2_l.T, xr2)
    return out
